# Initial kernel scaffold; baseline (speedup 1.0000x reference)
import jax
import jax.numpy as jnp
from jax.experimental import pallas as pl


def kernel(x, hw):
    raise NotImplementedError("write your pallas kernel here")


# speedup vs baseline: 8.7099x; 8.7099x over previous
import numpy as np
import ml_dtypes

import jax
import jax.numpy as jnp
from jax import lax
from jax.experimental import pallas as pl
from jax.experimental.pallas import tpu as pltpu
from jax.experimental.pallas import tpu_sc as plsc

_B, _D, _H, _W, _C = 4, 32, 96, 96, 32
_WC = _W * _C
_NBAND = 8
_BH = _H // _NBAND
_ROWS = _BH + 1
_NV = _WC // 16
_L = 16


def _axis_taps(n):
    c = np.linspace(-1.0, 1.0, n).astype(np.float32)
    c = c.astype(ml_dtypes.bfloat16).astype(np.float32)
    z = np.float32(0.5) * ((c + np.float32(1.0)) * np.float32(n - 2))
    z0 = np.clip(np.floor(z).astype(np.int32), 0, n - 1)
    f = (z - z0.astype(np.float32)).astype(np.float32)
    off = np.arange(n, dtype=np.int32) - z0
    assert set(np.unique(off)).issubset({0, 1, 2})
    return z0, f, off


_Z0D, _FD, _OFFD = _axis_taps(_D)
assert _OFFD[0] == 0 and _FD[0] == 0.0 and (_OFFD[1:] == 1).all()
_TD = np.concatenate([[np.float32(0.0)], np.float32(1.0) - _FD[1:]]).astype(np.float32)

_Z0H, _FH, _OFFH = _axis_taps(_H)
_H_OFF0 = [int(q) for q in np.nonzero(_OFFH == 0)[0]]
_H_OFF2 = [int(q) for q in np.nonzero(_OFFH == 2)[0]]

_Z0W, _FW, _OFFW = _axis_taps(_W)
_W_OFF0 = [int(q) for q in np.nonzero(_OFFW == 0)[0]]
_W_OFF2 = [int(q) for q in np.nonzero(_OFFW == 2)[0]]

_WT_H = _D * _L
_WT_W = _WT_H + _H * _L
_WTAB = np.concatenate([
    np.repeat(_TD, _L),
    np.repeat(_FH, _L),
    np.repeat(_FW[np.arange(_NV) >> 1], _L),
]).astype(np.float32)


def _sc_body(x_ref, w_ref, o_ref, bufa, bufb, obuf, wbuf):
    cid = lax.axis_index("c")
    sid = lax.axis_index("s")
    wid = sid * 2 + cid
    b = wid // _NBAND
    band = wid % _NBAND
    j0 = band * _BH
    off = jnp.minimum(j0, 1)
    start = j0 - off

    zv = jnp.zeros((_L,), jnp.float32)

    pltpu.sync_copy(w_ref, wbuf)

    def stage_off(i):
        return pl.multiple_of(((b * _D + i) * _H + start) * _WC, _WC)

    def process_slab(i, cur, prv):
        td = wbuf[pl.ds(i * _L, _L)]

        def dpass(n, _):
            v = n * _L
            xc = cur[pl.ds(v, _L)]
            xp = prv[pl.ds(v, _L)]
            prv[pl.ds(v, _L)] = xc + td * (xp - xc)
            return 0

        lax.fori_loop(0, _ROWS * _NV, dpass, 0, unroll=4)

        def row(jj, _):
            j = j0 + jj
            z0h = j - 1
            for q in _H_OFF0:
                z0h = z0h + (j == q).astype(jnp.int32)
            for q in _H_OFF2:
                z0h = z0h - (j == q).astype(jnp.int32)
            base_a = (z0h - start) * _WC
            base_b = base_a + _WC
            wh = wbuf[pl.ds(_WT_H + j * _L, _L)]
            obase = jj * _WC

            def hof(m):
                sa = prv[pl.ds(base_a + m * _L, _L)]
                sb = prv[pl.ds(base_b + m * _L, _L)]
                return sa + wh * (sb - sa)

            def vec(m, carry):
                h1, h2 = carry
                h = hof(m)
                ww = wbuf[pl.ds(_WT_W + m * _L, _L)]
                obuf[pl.ds(obase + m * _L, _L)] = h2 + ww * (h - h2)
                return (h, h1)

            lax.fori_loop(0, _NV, vec, (zv, zv), unroll=4)

            for k in _W_OFF0:
                fk = float(_FW[k])
                for par in (0, 1):
                    m = 2 * k + par
                    ha = hof(m)
                    hb = hof(m + 2)
                    obuf[pl.ds(obase + m * _L, _L)] = ha + fk * (hb - ha)
            for k in _W_OFF2:
                fk = float(_FW[k])
                for par in (0, 1):
                    m = 2 * k + par
                    ha = hof(m - 4)
                    hb = hof(m - 2)
                    obuf[pl.ds(obase + m * _L, _L)] = ha + fk * (hb - ha)
            return 0

        lax.fori_loop(0, _BH, row, 0)
        oofs = pl.multiple_of(((b * _D + i) * _H + j0) * _WC, _WC)
        pltpu.sync_copy(obuf, o_ref.at[pl.ds(oofs, _BH * _WC)])

    pltpu.sync_copy(x_ref.at[pl.ds(stage_off(jnp.int32(0)), _ROWS * _WC)], bufb)

    def outer(it, _):
        for p in range(2):
            i = it * 2 + p
            cur, prv = (bufa, bufb) if p == 0 else (bufb, bufa)
            pltpu.sync_copy(x_ref.at[pl.ds(stage_off(i), _ROWS * _WC)], cur)
            process_slab(i, cur, prv)
        return 0

    lax.fori_loop(0, _D // 2, outer, 0)


def kernel(x, hw):
    del hw
    x1 = x.reshape(_B * _D * _H * _WC)
    wt = jnp.asarray(_WTAB)
    mesh = plsc.VectorSubcoreMesh(core_axis_name="c", subcore_axis_name="s")
    run = pl.kernel(
        _sc_body,
        out_type=jax.ShapeDtypeStruct((_B * _D * _H * _WC,), jnp.float32),
        mesh=mesh,
        scratch_types=[
            pltpu.VMEM((_ROWS * _WC,), jnp.float32),
            pltpu.VMEM((_ROWS * _WC,), jnp.float32),
            pltpu.VMEM((_BH * _WC,), jnp.float32),
            pltpu.VMEM((_WTAB.size,), jnp.float32),
        ],
    )
    return run(x1, wt).reshape(_B, _D, _H, _W, _C)


# speedup vs baseline: 10.4349x; 1.1981x over previous
import numpy as np
import ml_dtypes

import jax
import jax.numpy as jnp
from jax import lax
from jax.experimental import pallas as pl
from jax.experimental.pallas import tpu as pltpu
from jax.experimental.pallas import tpu_sc as plsc

_B, _D, _H, _W, _C = 4, 32, 96, 96, 32
_WC = _W * _C
_NBAND = 8
_BH = _H // _NBAND
_ROWS = _BH + 1
_NV = _WC // 16
_L = 16


def _axis_taps(n):
    c = np.linspace(-1.0, 1.0, n).astype(np.float32)
    c = c.astype(ml_dtypes.bfloat16).astype(np.float32)
    z = np.float32(0.5) * ((c + np.float32(1.0)) * np.float32(n - 2))
    z0 = np.clip(np.floor(z).astype(np.int32), 0, n - 1)
    f = (z - z0.astype(np.float32)).astype(np.float32)
    off = np.arange(n, dtype=np.int32) - z0
    assert set(np.unique(off)).issubset({0, 1, 2})
    return z0, f, off


_Z0D, _FD, _OFFD = _axis_taps(_D)
assert _OFFD[0] == 0 and _FD[0] == 0.0 and (_OFFD[1:] == 1).all()
_TD = np.concatenate([[np.float32(0.0)], np.float32(1.0) - _FD[1:]]).astype(np.float32)

_Z0H, _FH, _OFFH = _axis_taps(_H)
_H_OFF0 = [int(q) for q in np.nonzero(_OFFH == 0)[0]]
_H_OFF2 = [int(q) for q in np.nonzero(_OFFH == 2)[0]]

_Z0W, _FW, _OFFW = _axis_taps(_W)
_W_OFF0 = [int(q) for q in np.nonzero(_OFFW == 0)[0]]
_W_OFF2 = [int(q) for q in np.nonzero(_OFFW == 2)[0]]

_WT_H = _D * _L
_WT_W = _WT_H + _H * _L
_WTAB = np.concatenate([
    np.repeat(_TD, _L),
    np.repeat(_FH, _L),
    np.repeat(_FW[np.arange(_NV) >> 1], _L),
]).astype(np.float32)


def _sc_body(x_ref, w_ref, o_ref, bufa, bufb, obuf, wbuf):
    cid = lax.axis_index("c")
    sid = lax.axis_index("s")
    wid = sid * 2 + cid
    b = wid // _NBAND
    band = wid % _NBAND
    j0 = band * _BH
    off = jnp.minimum(j0, 1)
    start = j0 - off

    zv = jnp.zeros((_L,), jnp.float32)

    pltpu.sync_copy(w_ref, wbuf)

    def stage_off(i):
        return pl.multiple_of(((b * _D + i) * _H + start) * _WC, _WC)

    def process_slab(i, cur, prv):
        td = wbuf[pl.ds(i * _L, _L)]

        @plsc.parallel_loop(0, _ROWS * _NV, unroll=8)
        def dpass(n):
            v = n * _L
            xc = cur[pl.ds(v, _L)]
            xp = prv[pl.ds(v, _L)]
            prv[pl.ds(v, _L)] = xc + td * (xp - xc)

        def row(jj, _):
            j = j0 + jj
            z0h = j - 1
            for q in _H_OFF0:
                z0h = z0h + (j == q).astype(jnp.int32)
            for q in _H_OFF2:
                z0h = z0h - (j == q).astype(jnp.int32)
            base_a = (z0h - start) * _WC
            base_b = base_a + _WC
            wh = wbuf[pl.ds(_WT_H + j * _L, _L)]
            obase = jj * _WC

            def hof(m):
                sa = prv[pl.ds(base_a + m * _L, _L)]
                sb = prv[pl.ds(base_b + m * _L, _L)]
                return sa + wh * (sb - sa)

            def vec(m, carry):
                h1, h2 = carry
                h = hof(m)
                ww = wbuf[pl.ds(_WT_W + m * _L, _L)]
                obuf[pl.ds(obase + m * _L, _L)] = h2 + ww * (h - h2)
                return (h, h1)

            lax.fori_loop(0, _NV, vec, (zv, zv), unroll=4)

            for k in _W_OFF0:
                fk = float(_FW[k])
                for par in (0, 1):
                    m = 2 * k + par
                    ha = hof(m)
                    hb = hof(m + 2)
                    obuf[pl.ds(obase + m * _L, _L)] = ha + fk * (hb - ha)
            for k in _W_OFF2:
                fk = float(_FW[k])
                for par in (0, 1):
                    m = 2 * k + par
                    ha = hof(m - 4)
                    hb = hof(m - 2)
                    obuf[pl.ds(obase + m * _L, _L)] = ha + fk * (hb - ha)
            return 0

        lax.fori_loop(0, _BH, row, 0)
        oofs = pl.multiple_of(((b * _D + i) * _H + j0) * _WC, _WC)
        pltpu.sync_copy(obuf, o_ref.at[pl.ds(oofs, _BH * _WC)])

    pltpu.sync_copy(x_ref.at[pl.ds(stage_off(jnp.int32(0)), _ROWS * _WC)], bufb)

    def outer(it, _):
        for p in range(2):
            i = it * 2 + p
            cur, prv = (bufa, bufb) if p == 0 else (bufb, bufa)
            pltpu.sync_copy(x_ref.at[pl.ds(stage_off(i), _ROWS * _WC)], cur)
            process_slab(i, cur, prv)
        return 0

    lax.fori_loop(0, _D // 2, outer, 0)


def kernel(x, hw):
    del hw
    x1 = x.reshape(_B * _D * _H * _WC)
    wt = jnp.asarray(_WTAB)
    mesh = plsc.VectorSubcoreMesh(core_axis_name="c", subcore_axis_name="s")
    run = pl.kernel(
        _sc_body,
        out_type=jax.ShapeDtypeStruct((_B * _D * _H * _WC,), jnp.float32),
        mesh=mesh,
        scratch_types=[
            pltpu.VMEM((_ROWS * _WC,), jnp.float32),
            pltpu.VMEM((_ROWS * _WC,), jnp.float32),
            pltpu.VMEM((_BH * _WC,), jnp.float32),
            pltpu.VMEM((_WTAB.size,), jnp.float32),
        ],
    )
    return run(x1, wt).reshape(_B, _D, _H, _W, _C)
